# Initial kernel scaffold; baseline (speedup 1.0000x reference)
import functools

import jax
import jax.numpy as jnp
from jax import lax
from jax.experimental import pallas as pl
from jax.experimental.pallas import tpu as pltpu
from jax.experimental.pallas import tpu_sc as plsc

N_NODES = 10000
N_EDGES = 320000
F = 128
NEG_SLOPE = 0.2

N_PAD = 10240
NC, NS = 2, 16
NW = NC * NS
CHUNK = 128
CPW = 80
E_PAD = NW * CPW * CHUNK
N_DUMMY_ROWS = N_PAD - N_NODES
ROWS_PER_TILE = N_PAD // NS


def _pre_body(x_ref, w_ref, asrc_ref, adst_ref, xp_ref, a1_ref, a2_ref):
    xp = jnp.dot(x_ref[...], w_ref[...], preferred_element_type=jnp.float32)
    xp_ref[...] = xp
    a1_ref[...] = jnp.sum(xp * asrc_ref[...], axis=1, keepdims=True)
    a2_ref[...] = jnp.sum(xp * adst_ref[...], axis=1, keepdims=True)


def _tc_pre(x_pad, W, att_src_row, att_dst_row):
    blk = 1024
    grid = N_PAD // blk
    return pl.pallas_call(
        _pre_body,
        grid=(grid,),
        in_specs=[
            pl.BlockSpec((blk, F), lambda i: (i, 0)),
            pl.BlockSpec((F, F), lambda i: (0, 0)),
            pl.BlockSpec((1, F), lambda i: (0, 0)),
            pl.BlockSpec((1, F), lambda i: (0, 0)),
        ],
        out_specs=[
            pl.BlockSpec((blk, F), lambda i: (i, 0)),
            pl.BlockSpec((blk, 1), lambda i: (i, 0)),
            pl.BlockSpec((blk, 1), lambda i: (i, 0)),
        ],
        out_shape=[
            jax.ShapeDtypeStruct((N_PAD, F), jnp.float32),
            jax.ShapeDtypeStruct((N_PAD, 1), jnp.float32),
            jax.ShapeDtypeStruct((N_PAD, 1), jnp.float32),
        ],
    )(x_pad, W, att_src_row, att_dst_row)


def _sc_edge_body(xp_hbm, asrc_hbm, adst_hbm, srcc_hbm, dstc_hbm,
                  acc_out, den_out,
                  asrc_v, adst_v, src_v, dst_v, rows_v, w_v, zrow_v, dz_v,
                  acc_sp, den_sp, sem):
    cid = lax.axis_index("c")
    sid = lax.axis_index("s")
    wid = sid * NC + cid

    pltpu.sync_copy(asrc_hbm, asrc_v)
    pltpu.sync_copy(adst_hbm, adst_v)
    pltpu.sync_copy(srcc_hbm.at[pl.ds(wid * CPW, CPW)], src_v)
    pltpu.sync_copy(dstc_hbm.at[pl.ds(wid * CPW, CPW)], dst_v)

    def _zrow(i, _):
        for j in range(F // 16):
            zrow_v[i, pl.ds(j * 16, 16)] = jnp.zeros((16,), jnp.float32)
        return 0
    lax.fori_loop(0, CHUNK, _zrow, 0)

    def _zden(i, _):
        dz_v[pl.ds(i * 16, 16)] = jnp.zeros((16,), jnp.float32)
        return 0
    lax.fori_loop(0, ROWS_PER_TILE // 16, _zden, 0)

    row0 = sid * ROWS_PER_TILE
    for k in range(ROWS_PER_TILE // CHUNK):
        pltpu.sync_copy(zrow_v, acc_sp.at[pl.ds(row0 + k * CHUNK, CHUNK)])
    pltpu.sync_copy(dz_v, den_sp.at[pl.ds(row0, ROWS_PER_TILE)])

    plsc.subcore_barrier()

    def _chunk(c, _):
        pltpu.async_copy(xp_hbm.at[src_v.at[c]], rows_v, sem).wait()
        for g in range(CHUNK // 16):
            sl = pl.ds(g * 16, 16)
            si = src_v[c, sl]
            di = dst_v[c, sl]
            s = plsc.load_gather(asrc_v, [si]) + plsc.load_gather(adst_v, [di])
            s = jnp.where(s > 0, s, s * NEG_SLOPE)
            w_v[sl] = jnp.exp(s)
        pltpu.sync_copy(w_v, den_sp.at[dst_v.at[c]], add=True)

        def _row(k, _):
            ws = w_v[k]
            for j in range(F // 16):
                sl = pl.ds(j * 16, 16)
                rows_v[k, sl] = rows_v[k, sl] * ws
            return 0
        lax.fori_loop(0, CHUNK, _row, 0)

        pltpu.sync_copy(rows_v, acc_sp.at[dst_v.at[c]], add=True)
        return 0

    lax.fori_loop(0, CPW, _chunk, 0)

    plsc.subcore_barrier()

    pltpu.sync_copy(acc_sp.at[pl.ds(row0, ROWS_PER_TILE)],
                    acc_out.at[cid, pl.ds(row0, ROWS_PER_TILE)])
    pltpu.sync_copy(den_sp.at[pl.ds(row0, ROWS_PER_TILE)],
                    den_out.at[cid, pl.ds(row0, ROWS_PER_TILE)])


def _sc_edge(xp, asrc_flat, adst_flat, src_chunks, dst_chunks):
    mesh = plsc.VectorSubcoreMesh(
        core_axis_name="c", subcore_axis_name="s",
        num_cores=NC, num_subcores=NS)
    fn = pl.kernel(
        _sc_edge_body,
        out_type=[
            jax.ShapeDtypeStruct((NC, N_PAD, F), jnp.float32),
            jax.ShapeDtypeStruct((NC, N_PAD), jnp.float32),
        ],
        mesh=mesh,
        scratch_types=[
            pltpu.VMEM((N_PAD,), jnp.float32),
            pltpu.VMEM((N_PAD,), jnp.float32),
            pltpu.VMEM((CPW, CHUNK), jnp.int32),
            pltpu.VMEM((CPW, CHUNK), jnp.int32),
            pltpu.VMEM((CHUNK, F), jnp.float32),
            pltpu.VMEM((CHUNK,), jnp.float32),
            pltpu.VMEM((CHUNK, F), jnp.float32),
            pltpu.VMEM((ROWS_PER_TILE,), jnp.float32),
            pltpu.VMEM_SHARED((N_PAD, F), jnp.float32),
            pltpu.VMEM_SHARED((N_PAD,), jnp.float32),
            pltpu.SemaphoreType.DMA,
        ],
    )
    return fn(xp, asrc_flat, adst_flat, src_chunks, dst_chunks)


def _post_body(acc_ref, den_ref, xp_ref, a1_ref, a2_ref, bias_ref, out_ref):
    acc = acc_ref[0] + acc_ref[1]
    den = den_ref[0] + den_ref[1]
    a = a1_ref[...] + a2_ref[...]
    a = jnp.where(a > 0, a, a * NEG_SLOPE)
    w_self = jnp.exp(a)
    num = acc + w_self * xp_ref[...]
    den = den + w_self + 1e-16
    out_ref[...] = num / den + bias_ref[...]


def _tc_post(acc, den3, xp, a1, a2, bias_row):
    blk = 1024
    grid = N_PAD // blk
    return pl.pallas_call(
        _post_body,
        grid=(grid,),
        in_specs=[
            pl.BlockSpec((NC, blk, F), lambda i: (0, i, 0)),
            pl.BlockSpec((NC, blk, 1), lambda i: (0, i, 0)),
            pl.BlockSpec((blk, F), lambda i: (i, 0)),
            pl.BlockSpec((blk, 1), lambda i: (i, 0)),
            pl.BlockSpec((blk, 1), lambda i: (i, 0)),
            pl.BlockSpec((1, F), lambda i: (0, 0)),
        ],
        out_specs=pl.BlockSpec((blk, F), lambda i: (i, 0)),
        out_shape=jax.ShapeDtypeStruct((N_PAD, F), jnp.float32),
    )(acc, den3, xp, a1, a2, bias_row)


@jax.jit
def kernel(x, edge_index, W, att_src, att_dst, bias):
    x_pad = jnp.pad(x, ((0, N_PAD - N_NODES), (0, 0)))
    att_src_row = att_src.reshape(1, F)
    att_dst_row = att_dst.reshape(1, F)
    bias_row = bias.reshape(1, F)

    xp, a1, a2 = _tc_pre(x_pad, W, att_src_row, att_dst_row)

    src = edge_index[0].astype(jnp.int32)
    dst = edge_index[1].astype(jnp.int32)
    n_dummy = E_PAD - N_EDGES
    pad_idx = N_NODES + (jnp.arange(n_dummy, dtype=jnp.int32) % N_DUMMY_ROWS)
    src_chunks = jnp.concatenate([src, pad_idx]).reshape(-1, CHUNK)
    dst_chunks = jnp.concatenate([dst, pad_idx]).reshape(-1, CHUNK)

    acc, den = _sc_edge(xp, a1.reshape(N_PAD), a2.reshape(N_PAD),
                        src_chunks, dst_chunks)

    out = _tc_post(acc, den.reshape(NC, N_PAD, 1), xp, a1, a2, bias_row)
    return out[:N_NODES]


# speedup vs baseline: 30.7290x; 30.7290x over previous
import functools

import jax
import jax.numpy as jnp
from jax import lax
from jax.experimental import pallas as pl
from jax.experimental.pallas import tpu as pltpu
from jax.experimental.pallas import tpu_sc as plsc

N_NODES = 10000
N_EDGES = 320000
F = 128
NEG_SLOPE = 0.2

N_PAD = 10240
NC, NS = 2, 16
NW = NC * NS
CHUNK = 64
CPW = 160
GRP = 16
E_PAD = NW * CPW * CHUNK
N_DUMMY_ROWS = N_PAD - N_NODES
ROWS_PER_TILE = N_PAD // NS


def _pre_body(x_ref, w_ref, asrc_ref, adst_ref, xp_ref, a1_ref, a2_ref):
    xp = jnp.dot(x_ref[...], w_ref[...], preferred_element_type=jnp.float32)
    xp_ref[...] = xp
    a1_ref[...] = jnp.sum(xp * asrc_ref[...], axis=1, keepdims=True)
    a2_ref[...] = jnp.sum(xp * adst_ref[...], axis=1, keepdims=True)


def _tc_pre(x_pad, W, att_src_row, att_dst_row):
    blk = 1024
    grid = N_PAD // blk
    return pl.pallas_call(
        _pre_body,
        grid=(grid,),
        in_specs=[
            pl.BlockSpec((blk, F), lambda i: (i, 0)),
            pl.BlockSpec((F, F), lambda i: (0, 0)),
            pl.BlockSpec((1, F), lambda i: (0, 0)),
            pl.BlockSpec((1, F), lambda i: (0, 0)),
        ],
        out_specs=[
            pl.BlockSpec((blk, F), lambda i: (i, 0)),
            pl.BlockSpec((blk, 1), lambda i: (i, 0)),
            pl.BlockSpec((blk, 1), lambda i: (i, 0)),
        ],
        out_shape=[
            jax.ShapeDtypeStruct((N_PAD, F), jnp.float32),
            jax.ShapeDtypeStruct((N_PAD, 1), jnp.float32),
            jax.ShapeDtypeStruct((N_PAD, 1), jnp.float32),
        ],
    )(x_pad, W, att_src_row, att_dst_row)


def _sc_edge_body(xp_hbm, asrc_hbm, adst_hbm, srcc_hbm, dstc_hbm,
                  acc_out, den_out,
                  asrc_v, adst_v, src_v, dst_v, rows_v, w_v, dz_v,
                  acc_sp, den_sp, sem):
    cid = lax.axis_index("c")
    sid = lax.axis_index("s")
    wid = sid * NC + cid

    pltpu.sync_copy(asrc_hbm, asrc_v)
    pltpu.sync_copy(adst_hbm, adst_v)

    def _zrow(i, _):
        for j in range(F // 16):
            rows_v[i, pl.ds(j * 16, 16)] = jnp.zeros((16,), jnp.float32)
        return 0
    lax.fori_loop(0, CHUNK, _zrow, 0)

    def _zden(i, _):
        dz_v[pl.ds(i * 16, 16)] = jnp.zeros((16,), jnp.float32)
        return 0
    lax.fori_loop(0, ROWS_PER_TILE // 16, _zden, 0)

    row0 = sid * ROWS_PER_TILE
    for k in range(ROWS_PER_TILE // CHUNK):
        pltpu.sync_copy(rows_v, acc_sp.at[pl.ds(row0 + k * CHUNK, CHUNK)])
    pltpu.sync_copy(dz_v, den_sp.at[pl.ds(row0, ROWS_PER_TILE)])

    plsc.subcore_barrier()

    def _grp(sgi, _):
        pltpu.sync_copy(srcc_hbm.at[pl.ds(wid * CPW + sgi * GRP, GRP)], src_v)
        pltpu.sync_copy(dstc_hbm.at[pl.ds(wid * CPW + sgi * GRP, GRP)], dst_v)

        def _chunk(c, _):
            pltpu.async_copy(xp_hbm.at[src_v.at[c]], rows_v, sem).wait()
            for g in range(CHUNK // 16):
                sl = pl.ds(g * 16, 16)
                si = src_v[c, sl]
                di = dst_v[c, sl]
                s = (plsc.load_gather(asrc_v, [si])
                     + plsc.load_gather(adst_v, [di]))
                s = jnp.where(s > 0, s, s * NEG_SLOPE)
                w_v[sl] = jnp.exp(s)
            pltpu.sync_copy(w_v, den_sp.at[dst_v.at[c]], add=True)

            def _rowgrp(g, _):
                wv = w_v[pl.ds(g * 16, 16)]
                for r in range(16):
                    ws = wv[r]
                    k = g * 16 + r
                    for j in range(F // 16):
                        sl = pl.ds(j * 16, 16)
                        rows_v[k, sl] = rows_v[k, sl] * ws
                return 0
            lax.fori_loop(0, CHUNK // 16, _rowgrp, 0)

            pltpu.sync_copy(rows_v, acc_sp.at[dst_v.at[c]], add=True)
            return 0

        lax.fori_loop(0, GRP, _chunk, 0)
        return 0

    lax.fori_loop(0, CPW // GRP, _grp, 0)

    plsc.subcore_barrier()

    pltpu.sync_copy(acc_sp.at[pl.ds(row0, ROWS_PER_TILE)],
                    acc_out.at[cid, pl.ds(row0, ROWS_PER_TILE)])
    pltpu.sync_copy(den_sp.at[pl.ds(row0, ROWS_PER_TILE)],
                    den_out.at[cid, pl.ds(row0, ROWS_PER_TILE)])


def _sc_edge(xp, asrc_flat, adst_flat, src_chunks, dst_chunks):
    mesh = plsc.VectorSubcoreMesh(
        core_axis_name="c", subcore_axis_name="s",
        num_cores=NC, num_subcores=NS)
    fn = pl.kernel(
        _sc_edge_body,
        out_type=[
            jax.ShapeDtypeStruct((NC, N_PAD, F), jnp.float32),
            jax.ShapeDtypeStruct((NC, N_PAD), jnp.float32),
        ],
        mesh=mesh,
        scratch_types=[
            pltpu.VMEM((N_PAD,), jnp.float32),
            pltpu.VMEM((N_PAD,), jnp.float32),
            pltpu.VMEM((GRP, CHUNK), jnp.int32),
            pltpu.VMEM((GRP, CHUNK), jnp.int32),
            pltpu.VMEM((CHUNK, F), jnp.float32),
            pltpu.VMEM((CHUNK,), jnp.float32),
            pltpu.VMEM((ROWS_PER_TILE,), jnp.float32),
            pltpu.VMEM_SHARED((N_PAD, F), jnp.float32),
            pltpu.VMEM_SHARED((N_PAD,), jnp.float32),
            pltpu.SemaphoreType.DMA,
        ],
        compiler_params=pltpu.CompilerParams(needs_layout_passes=False),
    )
    return fn(xp, asrc_flat, adst_flat, src_chunks, dst_chunks)


def _post_body(acc_ref, den_ref, xp_ref, a1_ref, a2_ref, bias_ref, out_ref):
    acc = acc_ref[0] + acc_ref[1]
    den = den_ref[0] + den_ref[1]
    a = a1_ref[...] + a2_ref[...]
    a = jnp.where(a > 0, a, a * NEG_SLOPE)
    w_self = jnp.exp(a)
    num = acc + w_self * xp_ref[...]
    den = den + w_self + 1e-16
    out_ref[...] = num / den + bias_ref[...]


def _tc_post(acc, den3, xp, a1, a2, bias_row):
    blk = 1024
    grid = N_PAD // blk
    return pl.pallas_call(
        _post_body,
        grid=(grid,),
        in_specs=[
            pl.BlockSpec((NC, blk, F), lambda i: (0, i, 0)),
            pl.BlockSpec((NC, blk, 1), lambda i: (0, i, 0)),
            pl.BlockSpec((blk, F), lambda i: (i, 0)),
            pl.BlockSpec((blk, 1), lambda i: (i, 0)),
            pl.BlockSpec((blk, 1), lambda i: (i, 0)),
            pl.BlockSpec((1, F), lambda i: (0, 0)),
        ],
        out_specs=pl.BlockSpec((blk, F), lambda i: (i, 0)),
        out_shape=jax.ShapeDtypeStruct((N_PAD, F), jnp.float32),
    )(acc, den3, xp, a1, a2, bias_row)


@jax.jit
def kernel(x, edge_index, W, att_src, att_dst, bias):
    x_pad = jnp.pad(x, ((0, N_PAD - N_NODES), (0, 0)))
    att_src_row = att_src.reshape(1, F)
    att_dst_row = att_dst.reshape(1, F)
    bias_row = bias.reshape(1, F)

    xp, a1, a2 = _tc_pre(x_pad, W, att_src_row, att_dst_row)

    src = edge_index[0].astype(jnp.int32)
    dst = edge_index[1].astype(jnp.int32)
    n_dummy = E_PAD - N_EDGES
    pad_idx = N_NODES + (jnp.arange(n_dummy, dtype=jnp.int32) % N_DUMMY_ROWS)
    src_chunks = jnp.concatenate([src, pad_idx]).reshape(-1, CHUNK)
    dst_chunks = jnp.concatenate([dst, pad_idx]).reshape(-1, CHUNK)

    acc, den = _sc_edge(xp, a1.reshape(N_PAD), a2.reshape(N_PAD),
                        src_chunks, dst_chunks)

    out = _tc_post(acc, den.reshape(NC, N_PAD, 1), xp, a1, a2, bias_row)
    return out[:N_NODES]


# speedup vs baseline: 47.6613x; 1.5510x over previous
import functools

import jax
import jax.numpy as jnp
from jax import lax
from jax.experimental import pallas as pl
from jax.experimental.pallas import tpu as pltpu
from jax.experimental.pallas import tpu_sc as plsc

N_NODES = 10000
N_EDGES = 320000
F = 128
NEG_SLOPE = 0.2

N_PAD = 10240
NC, NS = 2, 16
NW = NC * NS
CHUNK = 48
CPW = 216
GRP = 24
NGRP = CPW // GRP
E_PAD = NW * CPW * CHUNK
N_DUMMY_ROWS = N_PAD - N_NODES
ROWS_PER_TILE = N_PAD // NS


def _pre_body(x_ref, w_ref, asrc_ref, adst_ref, xp_ref, a1_ref, a2_ref):
    xp = jnp.dot(x_ref[...], w_ref[...], preferred_element_type=jnp.float32)
    xp_ref[...] = xp
    a1_ref[...] = jnp.sum(xp * asrc_ref[...], axis=1, keepdims=True)
    a2_ref[...] = jnp.sum(xp * adst_ref[...], axis=1, keepdims=True)


def _tc_pre(x_pad, W, att_src_row, att_dst_row):
    blk = 1024
    grid = N_PAD // blk
    return pl.pallas_call(
        _pre_body,
        grid=(grid,),
        in_specs=[
            pl.BlockSpec((blk, F), lambda i: (i, 0)),
            pl.BlockSpec((F, F), lambda i: (0, 0)),
            pl.BlockSpec((1, F), lambda i: (0, 0)),
            pl.BlockSpec((1, F), lambda i: (0, 0)),
        ],
        out_specs=[
            pl.BlockSpec((blk, F), lambda i: (i, 0)),
            pl.BlockSpec((blk, 1), lambda i: (i, 0)),
            pl.BlockSpec((blk, 1), lambda i: (i, 0)),
        ],
        out_shape=[
            jax.ShapeDtypeStruct((N_PAD, F), jnp.float32),
            jax.ShapeDtypeStruct((N_PAD, 1), jnp.float32),
            jax.ShapeDtypeStruct((N_PAD, 1), jnp.float32),
        ],
    )(x_pad, W, att_src_row, att_dst_row)


def _sc_edge_body(xp_hbm, asrc_hbm, adst_hbm, srcc_hbm, dstc_hbm,
                  acc_out, den_out,
                  asrc_v, adst_v, src_v, dst_v,
                  rows_a, rows_b, rows_c, w_a, w_b, w_c, dz_v,
                  acc_sp, den_sp,
                  gsem_a, gsem_b, gsem_c, ssem_a, ssem_b, ssem_c):
    cid = lax.axis_index("c")
    sid = lax.axis_index("s")
    wid = sid * NC + cid
    rows = (rows_a, rows_b, rows_c)
    wbuf = (w_a, w_b, w_c)
    gsem = (gsem_a, gsem_b, gsem_c)
    ssem = (ssem_a, ssem_b, ssem_c)

    pltpu.sync_copy(asrc_hbm, asrc_v)
    pltpu.sync_copy(adst_hbm, adst_v)

    def _zrow(i, _):
        for j in range(F // 16):
            rows_a[i, pl.ds(j * 16, 16)] = jnp.zeros((16,), jnp.float32)
        return 0
    lax.fori_loop(0, CHUNK, _zrow, 0)

    def _zden(i, _):
        dz_v[pl.ds(i * 16, 16)] = jnp.zeros((16,), jnp.float32)
        return 0
    lax.fori_loop(0, ROWS_PER_TILE // 16, _zden, 0)

    row0 = sid * ROWS_PER_TILE
    nfull = ROWS_PER_TILE // CHUNK
    for k in range(nfull):
        pltpu.sync_copy(rows_a, acc_sp.at[pl.ds(row0 + k * CHUNK, CHUNK)])
    rem = ROWS_PER_TILE - nfull * CHUNK
    if rem:
        pltpu.sync_copy(rows_a.at[pl.ds(0, rem)],
                        acc_sp.at[pl.ds(row0 + nfull * CHUNK, rem)])
    pltpu.sync_copy(dz_v, den_sp.at[pl.ds(row0, ROWS_PER_TILE)])

    plsc.subcore_barrier()

    def _grp(sgi, _):
        base = wid * CPW + sgi * GRP
        pltpu.sync_copy(srcc_hbm.at[pl.ds(base, GRP)], src_v)
        pltpu.sync_copy(dstc_hbm.at[pl.ds(base, GRP)], dst_v)

        for u in range(3):
            pltpu.async_copy(xp_hbm.at[src_v.at[u]], rows[u], gsem[u])

        def _iter(i, _):
            for u in range(3):
                cl = i * 3 + u
                pltpu.make_async_copy(
                    xp_hbm.at[src_v.at[cl]], rows[u], gsem[u]).wait()
                for g in range(CHUNK // 16):
                    sl = pl.ds(g * 16, 16)
                    si = src_v[cl, sl]
                    di = dst_v[cl, sl]
                    s = (plsc.load_gather(asrc_v, [si])
                         + plsc.load_gather(adst_v, [di]))
                    s = jnp.where(s > 0, s, s * NEG_SLOPE)
                    wbuf[u][sl] = jnp.exp(s)

                def _rowgrp(g, _):
                    w16 = wbuf[u][pl.ds(g * 16, 16)]
                    for r in range(16):
                        ws = w16[r]
                        k = g * 16 + r
                        for j in range(F // 16):
                            sl = pl.ds(j * 16, 16)
                            rows[u][k, sl] = rows[u][k, sl] * ws
                    return 0
                lax.fori_loop(0, CHUNK // 16, _rowgrp, 0)

                pltpu.async_copy(wbuf[u], den_sp.at[dst_v.at[cl]],
                                 ssem[u], add=True)
                pltpu.async_copy(rows[u], acc_sp.at[dst_v.at[cl]],
                                 ssem[u], add=True)

                z = (u + 2) % 3

                def _advance(cl=cl, u=u, z=z):
                    clm1 = cl - 1
                    pltpu.make_async_copy(
                        wbuf[z], den_sp.at[dst_v.at[clm1]], ssem[z]).wait()
                    pltpu.make_async_copy(
                        rows[z], acc_sp.at[dst_v.at[clm1]], ssem[z]).wait()
                    pltpu.async_copy(
                        xp_hbm.at[src_v.at[cl + 2]], rows[z], gsem[z])

                if u == 0:
                    pl.when(i >= 1)(_advance)
                else:
                    pl.when(i <= GRP // 3 - 2)(_advance)
            return 0

        lax.fori_loop(0, GRP // 3, _iter, 0)

        for u in range(3):
            cl = GRP - 3 + u
            pltpu.make_async_copy(
                wbuf[u], den_sp.at[dst_v.at[cl]], ssem[u]).wait()
            pltpu.make_async_copy(
                rows[u], acc_sp.at[dst_v.at[cl]], ssem[u]).wait()
        return 0

    lax.fori_loop(0, NGRP, _grp, 0)

    plsc.subcore_barrier()

    pltpu.sync_copy(acc_sp.at[pl.ds(row0, ROWS_PER_TILE)],
                    acc_out.at[cid, pl.ds(row0, ROWS_PER_TILE)])
    pltpu.sync_copy(den_sp.at[pl.ds(row0, ROWS_PER_TILE)],
                    den_out.at[cid, pl.ds(row0, ROWS_PER_TILE)])


def _sc_edge(xp, asrc_flat, adst_flat, src_chunks, dst_chunks):
    mesh = plsc.VectorSubcoreMesh(
        core_axis_name="c", subcore_axis_name="s",
        num_cores=NC, num_subcores=NS)
    fn = pl.kernel(
        _sc_edge_body,
        out_type=[
            jax.ShapeDtypeStruct((NC, N_PAD, F), jnp.float32),
            jax.ShapeDtypeStruct((NC, N_PAD), jnp.float32),
        ],
        mesh=mesh,
        scratch_types=[
            pltpu.VMEM((N_PAD,), jnp.float32),
            pltpu.VMEM((N_PAD,), jnp.float32),
            pltpu.VMEM((GRP, CHUNK), jnp.int32),
            pltpu.VMEM((GRP, CHUNK), jnp.int32),
            pltpu.VMEM((CHUNK, F), jnp.float32),
            pltpu.VMEM((CHUNK, F), jnp.float32),
            pltpu.VMEM((CHUNK, F), jnp.float32),
            pltpu.VMEM((CHUNK,), jnp.float32),
            pltpu.VMEM((CHUNK,), jnp.float32),
            pltpu.VMEM((CHUNK,), jnp.float32),
            pltpu.VMEM((ROWS_PER_TILE,), jnp.float32),
            pltpu.VMEM_SHARED((N_PAD, F), jnp.float32),
            pltpu.VMEM_SHARED((N_PAD,), jnp.float32),
            pltpu.SemaphoreType.DMA,
            pltpu.SemaphoreType.DMA,
            pltpu.SemaphoreType.DMA,
            pltpu.SemaphoreType.DMA,
            pltpu.SemaphoreType.DMA,
            pltpu.SemaphoreType.DMA,
        ],
        compiler_params=pltpu.CompilerParams(needs_layout_passes=False),
    )
    return fn(xp, asrc_flat, adst_flat, src_chunks, dst_chunks)


def _post_body(acc_ref, den_ref, xp_ref, a1_ref, a2_ref, bias_ref, out_ref):
    acc = acc_ref[0] + acc_ref[1]
    den = den_ref[0] + den_ref[1]
    a = a1_ref[...] + a2_ref[...]
    a = jnp.where(a > 0, a, a * NEG_SLOPE)
    w_self = jnp.exp(a)
    num = acc + w_self * xp_ref[...]
    den = den + w_self + 1e-16
    out_ref[...] = num / den + bias_ref[...]


def _tc_post(acc, den3, xp, a1, a2, bias_row):
    blk = 1024
    grid = N_PAD // blk
    return pl.pallas_call(
        _post_body,
        grid=(grid,),
        in_specs=[
            pl.BlockSpec((NC, blk, F), lambda i: (0, i, 0)),
            pl.BlockSpec((NC, blk, 1), lambda i: (0, i, 0)),
            pl.BlockSpec((blk, F), lambda i: (i, 0)),
            pl.BlockSpec((blk, 1), lambda i: (i, 0)),
            pl.BlockSpec((blk, 1), lambda i: (i, 0)),
            pl.BlockSpec((1, F), lambda i: (0, 0)),
        ],
        out_specs=pl.BlockSpec((blk, F), lambda i: (i, 0)),
        out_shape=jax.ShapeDtypeStruct((N_PAD, F), jnp.float32),
    )(acc, den3, xp, a1, a2, bias_row)


@jax.jit
def kernel(x, edge_index, W, att_src, att_dst, bias):
    x_pad = jnp.pad(x, ((0, N_PAD - N_NODES), (0, 0)))
    att_src_row = att_src.reshape(1, F)
    att_dst_row = att_dst.reshape(1, F)
    bias_row = bias.reshape(1, F)

    xp, a1, a2 = _tc_pre(x_pad, W, att_src_row, att_dst_row)

    src = edge_index[0].astype(jnp.int32)
    dst = edge_index[1].astype(jnp.int32)
    n_dummy = E_PAD - N_EDGES
    pad_idx = N_NODES + (jnp.arange(n_dummy, dtype=jnp.int32) % N_DUMMY_ROWS)
    src_chunks = jnp.concatenate([src, pad_idx]).reshape(-1, CHUNK)
    dst_chunks = jnp.concatenate([dst, pad_idx]).reshape(-1, CHUNK)

    acc, den = _sc_edge(xp, a1.reshape(N_PAD), a2.reshape(N_PAD),
                        src_chunks, dst_chunks)

    out = _tc_post(acc, den.reshape(NC, N_PAD, 1), xp, a1, a2, bias_row)
    return out[:N_NODES]


# speedup vs baseline: 47.8930x; 1.0049x over previous
import functools

import jax
import jax.numpy as jnp
from jax import lax
from jax.experimental import pallas as pl
from jax.experimental.pallas import tpu as pltpu
from jax.experimental.pallas import tpu_sc as plsc

N_NODES = 10000
N_EDGES = 320000
F = 128
NEG_SLOPE = 0.2

N_PAD = 10240
NC, NS = 2, 16
NW = NC * NS
CHUNK = 48
CPW = 216
GRP = 24
NGRP = CPW // GRP
E_PAD = NW * CPW * CHUNK
N_DUMMY_ROWS = N_PAD - N_NODES
ROWS_PER_TILE = N_PAD // NS


def _pre_body(x_ref, w_ref, asrc_ref, adst_ref, xp_ref, a1_ref, a2_ref):
    xp = jnp.dot(x_ref[...], w_ref[...], preferred_element_type=jnp.float32)
    xp_ref[...] = xp
    a1_ref[...] = jnp.sum(xp * asrc_ref[...], axis=1, keepdims=True)
    a2_ref[...] = jnp.sum(xp * adst_ref[...], axis=1, keepdims=True)


def _tc_pre(x_pad, W, att_src_row, att_dst_row):
    blk = 1024
    grid = N_PAD // blk
    return pl.pallas_call(
        _pre_body,
        grid=(grid,),
        in_specs=[
            pl.BlockSpec((blk, F), lambda i: (i, 0)),
            pl.BlockSpec((F, F), lambda i: (0, 0)),
            pl.BlockSpec((1, F), lambda i: (0, 0)),
            pl.BlockSpec((1, F), lambda i: (0, 0)),
        ],
        out_specs=[
            pl.BlockSpec((blk, F), lambda i: (i, 0)),
            pl.BlockSpec((blk, 1), lambda i: (i, 0)),
            pl.BlockSpec((blk, 1), lambda i: (i, 0)),
        ],
        out_shape=[
            jax.ShapeDtypeStruct((N_PAD, F), jnp.float32),
            jax.ShapeDtypeStruct((N_PAD, 1), jnp.float32),
            jax.ShapeDtypeStruct((N_PAD, 1), jnp.float32),
        ],
    )(x_pad, W, att_src_row, att_dst_row)


def _sc_edge_body(xp_hbm, asrc_hbm, adst_hbm, srcc_hbm, dstc_hbm,
                  acc_out, den_out,
                  asrc_v, adst_v, src_v, dst_v,
                  rows_a, rows_b, rows_c, w_a, w_b, w_c, dz_v,
                  acc_sp, den_sp,
                  gsem_a, gsem_b, gsem_c, ssem_a, ssem_b, ssem_c):
    cid = lax.axis_index("c")
    sid = lax.axis_index("s")
    wid = sid * NC + cid
    rows = (rows_a, rows_b, rows_c)
    wbuf = (w_a, w_b, w_c)
    gsem = (gsem_a, gsem_b, gsem_c)
    ssem = (ssem_a, ssem_b, ssem_c)

    pltpu.sync_copy(asrc_hbm, asrc_v)
    pltpu.sync_copy(adst_hbm, adst_v)

    def _zrow(i, _):
        for j in range(F // 16):
            rows_a[i, pl.ds(j * 16, 16)] = jnp.zeros((16,), jnp.float32)
        return 0
    lax.fori_loop(0, CHUNK, _zrow, 0)

    def _zden(i, _):
        dz_v[pl.ds(i * 16, 16)] = jnp.zeros((16,), jnp.float32)
        return 0
    lax.fori_loop(0, ROWS_PER_TILE // 16, _zden, 0)

    row0 = sid * ROWS_PER_TILE
    nfull = ROWS_PER_TILE // CHUNK
    for k in range(nfull):
        pltpu.sync_copy(rows_a, acc_sp.at[pl.ds(row0 + k * CHUNK, CHUNK)])
    rem = ROWS_PER_TILE - nfull * CHUNK
    if rem:
        pltpu.sync_copy(rows_a.at[pl.ds(0, rem)],
                        acc_sp.at[pl.ds(row0 + nfull * CHUNK, rem)])
    pltpu.sync_copy(dz_v, den_sp.at[pl.ds(row0, ROWS_PER_TILE)])

    plsc.subcore_barrier()

    def _grp(sgi, _):
        base = wid * CPW + sgi * GRP
        pltpu.sync_copy(srcc_hbm.at[pl.ds(base, GRP)], src_v)
        pltpu.sync_copy(dstc_hbm.at[pl.ds(base, GRP)], dst_v)

        for u in range(3):
            pltpu.async_copy(xp_hbm.at[src_v.at[u]], rows[u], gsem[u])

        def _iter(i, _):
            for u in range(3):
                cl = i * 3 + u
                pltpu.make_async_copy(
                    xp_hbm.at[src_v.at[cl]], rows[u], gsem[u]).wait()
                for g in range(CHUNK // 16):
                    sl = pl.ds(g * 16, 16)
                    si = src_v[cl, sl]
                    di = dst_v[cl, sl]
                    s = (plsc.load_gather(asrc_v, [si])
                         + plsc.load_gather(adst_v, [di]))
                    s = jnp.where(s > 0, s, s * NEG_SLOPE)
                    wbuf[u][sl] = jnp.exp(s)

                for g in range(CHUNK // 16):
                    w16 = wbuf[u][pl.ds(g * 16, 16)]
                    for r in range(16):
                        ws = w16[r]
                        k = g * 16 + r
                        for j in range(F // 16):
                            sl = pl.ds(j * 16, 16)
                            rows[u][k, sl] = rows[u][k, sl] * ws

                pltpu.async_copy(wbuf[u], den_sp.at[dst_v.at[cl]],
                                 ssem[u], add=True)
                pltpu.async_copy(rows[u], acc_sp.at[dst_v.at[cl]],
                                 ssem[u], add=True)

                z = (u + 2) % 3

                def _advance(cl=cl, u=u, z=z):
                    clm1 = cl - 1
                    pltpu.make_async_copy(
                        wbuf[z], den_sp.at[dst_v.at[clm1]], ssem[z]).wait()
                    pltpu.make_async_copy(
                        rows[z], acc_sp.at[dst_v.at[clm1]], ssem[z]).wait()
                    pltpu.async_copy(
                        xp_hbm.at[src_v.at[cl + 2]], rows[z], gsem[z])

                if u == 0:
                    pl.when(i >= 1)(_advance)
                else:
                    pl.when(i <= GRP // 3 - 2)(_advance)
            return 0

        lax.fori_loop(0, GRP // 3, _iter, 0)

        for u in range(3):
            cl = GRP - 3 + u
            pltpu.make_async_copy(
                wbuf[u], den_sp.at[dst_v.at[cl]], ssem[u]).wait()
            pltpu.make_async_copy(
                rows[u], acc_sp.at[dst_v.at[cl]], ssem[u]).wait()
        return 0

    lax.fori_loop(0, NGRP, _grp, 0)

    plsc.subcore_barrier()

    pltpu.sync_copy(acc_sp.at[pl.ds(row0, ROWS_PER_TILE)],
                    acc_out.at[cid, pl.ds(row0, ROWS_PER_TILE)])
    pltpu.sync_copy(den_sp.at[pl.ds(row0, ROWS_PER_TILE)],
                    den_out.at[cid, pl.ds(row0, ROWS_PER_TILE)])


def _sc_edge(xp, asrc_flat, adst_flat, src_chunks, dst_chunks):
    mesh = plsc.VectorSubcoreMesh(
        core_axis_name="c", subcore_axis_name="s",
        num_cores=NC, num_subcores=NS)
    fn = pl.kernel(
        _sc_edge_body,
        out_type=[
            jax.ShapeDtypeStruct((NC, N_PAD, F), jnp.float32),
            jax.ShapeDtypeStruct((NC, N_PAD), jnp.float32),
        ],
        mesh=mesh,
        scratch_types=[
            pltpu.VMEM((N_PAD,), jnp.float32),
            pltpu.VMEM((N_PAD,), jnp.float32),
            pltpu.VMEM((GRP, CHUNK), jnp.int32),
            pltpu.VMEM((GRP, CHUNK), jnp.int32),
            pltpu.VMEM((CHUNK, F), jnp.float32),
            pltpu.VMEM((CHUNK, F), jnp.float32),
            pltpu.VMEM((CHUNK, F), jnp.float32),
            pltpu.VMEM((CHUNK,), jnp.float32),
            pltpu.VMEM((CHUNK,), jnp.float32),
            pltpu.VMEM((CHUNK,), jnp.float32),
            pltpu.VMEM((ROWS_PER_TILE,), jnp.float32),
            pltpu.VMEM_SHARED((N_PAD, F), jnp.float32),
            pltpu.VMEM_SHARED((N_PAD,), jnp.float32),
            pltpu.SemaphoreType.DMA,
            pltpu.SemaphoreType.DMA,
            pltpu.SemaphoreType.DMA,
            pltpu.SemaphoreType.DMA,
            pltpu.SemaphoreType.DMA,
            pltpu.SemaphoreType.DMA,
        ],
        compiler_params=pltpu.CompilerParams(needs_layout_passes=False),
    )
    return fn(xp, asrc_flat, adst_flat, src_chunks, dst_chunks)


def _post_body(acc_ref, den_ref, xp_ref, a1_ref, a2_ref, bias_ref, out_ref):
    acc = acc_ref[0] + acc_ref[1]
    den = den_ref[0] + den_ref[1]
    a = a1_ref[...] + a2_ref[...]
    a = jnp.where(a > 0, a, a * NEG_SLOPE)
    w_self = jnp.exp(a)
    num = acc + w_self * xp_ref[...]
    den = den + w_self + 1e-16
    out_ref[...] = num / den + bias_ref[...]


def _tc_post(acc, den3, xp, a1, a2, bias_row):
    blk = 1024
    grid = N_PAD // blk
    return pl.pallas_call(
        _post_body,
        grid=(grid,),
        in_specs=[
            pl.BlockSpec((NC, blk, F), lambda i: (0, i, 0)),
            pl.BlockSpec((NC, blk, 1), lambda i: (0, i, 0)),
            pl.BlockSpec((blk, F), lambda i: (i, 0)),
            pl.BlockSpec((blk, 1), lambda i: (i, 0)),
            pl.BlockSpec((blk, 1), lambda i: (i, 0)),
            pl.BlockSpec((1, F), lambda i: (0, 0)),
        ],
        out_specs=pl.BlockSpec((blk, F), lambda i: (i, 0)),
        out_shape=jax.ShapeDtypeStruct((N_PAD, F), jnp.float32),
    )(acc, den3, xp, a1, a2, bias_row)


@jax.jit
def kernel(x, edge_index, W, att_src, att_dst, bias):
    x_pad = jnp.pad(x, ((0, N_PAD - N_NODES), (0, 0)))
    att_src_row = att_src.reshape(1, F)
    att_dst_row = att_dst.reshape(1, F)
    bias_row = bias.reshape(1, F)

    xp, a1, a2 = _tc_pre(x_pad, W, att_src_row, att_dst_row)

    src = edge_index[0].astype(jnp.int32)
    dst = edge_index[1].astype(jnp.int32)
    n_dummy = E_PAD - N_EDGES
    pad_idx = N_NODES + (jnp.arange(n_dummy, dtype=jnp.int32) % N_DUMMY_ROWS)
    src_chunks = jnp.concatenate([src, pad_idx]).reshape(-1, CHUNK)
    dst_chunks = jnp.concatenate([dst, pad_idx]).reshape(-1, CHUNK)

    acc, den = _sc_edge(xp, a1.reshape(N_PAD), a2.reshape(N_PAD),
                        src_chunks, dst_chunks)

    out = _tc_post(acc, den.reshape(NC, N_PAD, 1), xp, a1, a2, bias_row)
    return out[:N_NODES]
